# baseline (speedup 1.0000x reference)
import functools

import jax
import jax.numpy as jnp
from jax import lax
from jax.experimental import pallas as pl
from jax.experimental.pallas import tpu as pltpu
from jax.experimental.pallas import tpu_sc as plsc

_F32 = jnp.float32


def _sigmoid16(x):
    return 1.0 / (1.0 + jnp.exp(-x))


def _tanh16(x):
    return 2.0 / (1.0 + jnp.exp(-2.0 * x)) - 1.0


def _prep_call(emb, wih, bih2):
    V, D = emb.shape
    G3 = wih.shape[1]

    def body(emb_ref, wih_ref, bih_ref, ea_ref, eb_ref):
        e = emb_ref[...]
        ea_ref[...] = jnp.dot(e, wih_ref[:D, :], preferred_element_type=_F32)
        eb_ref[...] = (jnp.dot(e, wih_ref[D:, :], preferred_element_type=_F32)
                       + bih_ref[...])

    return pl.pallas_call(
        body,
        out_shape=(jax.ShapeDtypeStruct((V, G3), _F32),
                   jax.ShapeDtypeStruct((V, G3), _F32)),
    )(emb, wih, bih2)


def _gather_call(ea, eb, emb, ids_pad, rids_pad):
    V, G3 = ea.shape
    D = emb.shape[1]
    NP = ids_pad.shape[0]
    NR = rids_pad.shape[0]
    NW = 32
    BPW = NP // NW
    CH = 80
    NCH = BPW // CH
    RPW = NR // 16
    mesh = plsc.VectorSubcoreMesh(core_axis_name="c", subcore_axis_name="s",
                                  num_cores=2)

    @functools.partial(
        pl.kernel,
        out_type=(jax.ShapeDtypeStruct((NP, G3), _F32),
                  jax.ShapeDtypeStruct((NP, G3), _F32),
                  jax.ShapeDtypeStruct((NR, D), _F32)),
        mesh=mesh,
        scratch_types=[
            pltpu.VMEM((CH,), jnp.int32),
            pltpu.VMEM((CH, G3), _F32),
            pltpu.VMEM((RPW,), jnp.int32),
            pltpu.VMEM((RPW, D), _F32),
            pltpu.SemaphoreType.DMA,
        ],
    )
    def k(ea_hbm, eb_hbm, emb_hbm, ids_hbm, rids_hbm, ga_hbm, gb_hbm, fr_hbm,
          idx_v, rows_v, ridx_v, rrows_v, sem):
        cid = lax.axis_index("c")
        sid = lax.axis_index("s")
        wid = cid * 16 + sid

        def chunk(i, carry):
            base = wid * BPW + i * CH
            pltpu.sync_copy(ids_hbm.at[pl.ds(base, CH)], idx_v)
            pltpu.async_copy(ea_hbm.at[idx_v], rows_v, sem).wait()
            pltpu.sync_copy(rows_v, ga_hbm.at[pl.ds(base, CH)])
            pltpu.async_copy(eb_hbm.at[idx_v], rows_v, sem).wait()
            pltpu.sync_copy(rows_v, gb_hbm.at[pl.ds(base, CH)])
            return carry

        lax.fori_loop(0, NCH, chunk, 0)

        @pl.when(cid == 0)
        def _():
            rbase = sid * RPW
            pltpu.sync_copy(rids_hbm.at[pl.ds(rbase, RPW)], ridx_v)
            pltpu.async_copy(emb_hbm.at[ridx_v], rrows_v, sem).wait()
            pltpu.sync_copy(rrows_v, fr_hbm.at[pl.ds(rbase, RPW)])

    return k(ea, eb, emb, ids_pad, rids_pad)


def _iteru_call(p, ga, whh, bhh2):
    N, G3 = ga.shape
    H = whh.shape[0]
    R = 2000

    def body(p_ref, ga_ref, whh_ref, bhh_ref, u_ref):
        s = p_ref[0] + p_ref[1]
        g = jnp.dot(s, whh_ref[...], preferred_element_type=_F32) + bhh_ref[...]
        ga_b = ga_ref[...]
        u_ref[...] = jnp.concatenate(
            [ga_b[:, :2 * H] + g[:, :2 * H], ga_b[:, 2 * H:], g[:, 2 * H:], s],
            axis=1)

    return pl.pallas_call(
        body,
        grid=(N // R,),
        in_specs=[
            pl.BlockSpec((2, R, H), lambda i: (0, i, 0)),
            pl.BlockSpec((R, G3), lambda i: (i, 0)),
            pl.BlockSpec((H, G3), lambda i: (0, 0)),
            pl.BlockSpec((1, G3), lambda i: (0, 0)),
        ],
        out_specs=pl.BlockSpec((R, G3 + 2 * H), lambda i: (i, 0)),
        out_shape=jax.ShapeDtypeStruct((N, G3 + 2 * H), _F32),
    )(p, ga, whh, bhh2)


def _edge_call(u, gb, src, dst, zero_nh):
    N, UW = u.shape
    G3 = gb.shape[1]
    E = src.shape[0]
    H = zero_nh.shape[1]
    NW = 32
    K = 40
    EPW = E // NW
    NB = EPW // K
    RPT = N // 16
    mesh = plsc.VectorSubcoreMesh(core_axis_name="c", subcore_axis_name="s",
                                  num_cores=2)

    @functools.partial(
        pl.kernel,
        out_type=jax.ShapeDtypeStruct((2, N, H), _F32),
        mesh=mesh,
        scratch_types=[
            pltpu.VMEM_SHARED((N, H), _F32),
            pltpu.VMEM((K,), jnp.int32),
            pltpu.VMEM((K,), jnp.int32),
            pltpu.VMEM((K, UW), _F32),
            pltpu.VMEM((K, G3), _F32),
            pltpu.VMEM((K, H), _F32),
            pltpu.SemaphoreType.DMA,
            pltpu.SemaphoreType.DMA,
        ],
    )
    def k(u_hbm, gb_hbm, src_hbm, dst_hbm, z_hbm, p_hbm,
          ssum, src_v, dst_v, u_rows, b_rows, m_buf, sem1, sem2):
        cid = lax.axis_index("c")
        sid = lax.axis_index("s")
        wid = cid * 16 + sid

        @pl.when(sid == 0)
        def _():
            pltpu.sync_copy(z_hbm, ssum)

        plsc.subcore_barrier()

        def block(i, carry):
            base = wid * EPW + i * K
            pltpu.sync_copy(src_hbm.at[pl.ds(base, K)], src_v)
            pltpu.sync_copy(dst_hbm.at[pl.ds(base, K)], dst_v)
            c1 = pltpu.async_copy(u_hbm.at[src_v], u_rows, sem1)
            c2 = pltpu.async_copy(gb_hbm.at[dst_v], b_rows, sem2)
            c1.wait()
            c2.wait()

            def edge(e, ecarry):
                for j in range(H // 16):
                    lo = j * 16
                    r = _sigmoid16(u_rows[e, pl.ds(lo, 16)]
                                   + b_rows[e, pl.ds(lo, 16)])
                    z = _sigmoid16(u_rows[e, pl.ds(H + lo, 16)]
                                   + b_rows[e, pl.ds(H + lo, 16)])
                    nn = _tanh16(u_rows[e, pl.ds(2 * H + lo, 16)]
                                 + b_rows[e, pl.ds(2 * H + lo, 16)]
                                 + r * u_rows[e, pl.ds(3 * H + lo, 16)])
                    m_buf[e, pl.ds(lo, 16)] = ((1.0 - z) * nn
                                               + z * u_rows[e, pl.ds(4 * H + lo, 16)])
                return ecarry

            lax.fori_loop(0, K, edge, 0)
            pltpu.sync_copy(m_buf, ssum.at[dst_v], add=True)
            return carry

        lax.fori_loop(0, NB, block, 0)
        plsc.subcore_barrier()

        @pl.when(sid == 0)
        def _():
            pltpu.sync_copy(ssum, p_hbm.at[cid])

    return k(u, gb, src, dst, zero_nh)


def _final_call(fr, pr):
    R, D = fr.shape
    H = pr.shape[2]

    def body(fr_ref, pr_ref, o_ref):
        o_ref[...] = jnp.concatenate([fr_ref[...], pr_ref[0] + pr_ref[1]],
                                     axis=1)

    return pl.pallas_call(
        body,
        out_shape=jax.ShapeDtypeStruct((R, D + H), _F32),
    )(fr, pr)


def kernel(ids, edge_index, depth, embeddings, W_ih, W_hh, b_ih, b_hh):
    N = ids.shape[0]
    V, D = embeddings.shape
    H = W_hh.shape[0]
    src = edge_index[0].astype(jnp.int32)
    dst = edge_index[1].astype(jnp.int32)

    ea, eb = _prep_call(embeddings, W_ih, b_ih.reshape(1, -1))

    ids32 = ids.astype(jnp.int32)
    NP = 10240
    ids_pad = jnp.concatenate([ids32, jnp.zeros((NP - N,), jnp.int32)])
    rids = ids32.reshape(100, 100)[:, 0]
    rids_pad = jnp.concatenate([rids, jnp.zeros((28,), jnp.int32)])
    ga_p, gb_p, fr = _gather_call(ea, eb, embeddings, ids_pad, rids_pad)
    ga = ga_p[:N]
    gb = gb_p[:N]

    zero_nh = jnp.zeros((N, H), _F32)
    p0 = jnp.zeros((2, N, H), _F32)
    bhh2 = b_hh.reshape(1, -1)

    def body(_, p):
        uu = _iteru_call(p, ga, W_hh, bhh2)
        return _edge_call(uu, gb, src, dst, zero_nh)

    p = lax.fori_loop(0, depth, body, p0)

    pr = p[:, ::100, :]
    return _final_call(fr[:100], pr)


# speedup vs baseline: 2.2488x; 2.2488x over previous
import functools

import jax
import jax.numpy as jnp
from jax import lax
from jax.experimental import pallas as pl
from jax.experimental.pallas import tpu as pltpu
from jax.experimental.pallas import tpu_sc as plsc

_F32 = jnp.float32


def _sigmoid16(x):
    return 1.0 / (1.0 + jnp.exp(-x))


def _tanh16(x):
    return 2.0 / (1.0 + jnp.exp(-2.0 * x)) - 1.0


def _pack_bf16_cols(x):
    ncol = x.shape[1]
    G = ncol // 32
    bits = jax.lax.bitcast_convert_type(x, jnp.int32)
    parts = []
    for g in range(G):
        lo = bits[:, 32 * g:32 * g + 16]
        hi = bits[:, 32 * g + 16:32 * g + 32]
        lo16 = jax.lax.shift_right_logical(lo + 0x8000, 16)
        hi16 = (hi + 0x8000) & jnp.int32(-65536)
        parts.append(lo16 | hi16)
    return jnp.concatenate(parts, axis=1)


def _unpk(w):
    lo = jax.lax.bitcast_convert_type(w << 16, _F32)
    hi = jax.lax.bitcast_convert_type(w & jnp.int32(-65536), _F32)
    return lo, hi


def _prep_call(emb, wih, bih2):
    V, D = emb.shape
    G3 = wih.shape[1]

    def body(emb_ref, wih_ref, bih_ref, ea_ref, ebp_ref):
        e = emb_ref[...]
        ea_ref[...] = jnp.dot(e, wih_ref[:D, :], preferred_element_type=_F32)
        eb = (jnp.dot(e, wih_ref[D:, :], preferred_element_type=_F32)
              + bih_ref[...])
        packed = _pack_bf16_cols(eb)
        pad = jnp.zeros((packed.shape[0], 64), jnp.int32)
        ebp_ref[...] = jnp.concatenate([packed, pad], axis=1)

    return pl.pallas_call(
        body,
        out_shape=(jax.ShapeDtypeStruct((V, G3), _F32),
                   jax.ShapeDtypeStruct((V, G3 // 2 + 64), jnp.int32)),
    )(emb, wih, bih2)


def _gather_call(ea, ebp, emb, ids_pad, rids_pad):
    V, G3 = ea.shape
    GP = ebp.shape[1]
    D = emb.shape[1]
    NP = ids_pad.shape[0]
    NR = rids_pad.shape[0]
    NW = 32
    BPW = NP // NW
    CH = 80
    NCH = BPW // CH
    RPW = NR // 16
    mesh = plsc.VectorSubcoreMesh(core_axis_name="c", subcore_axis_name="s",
                                  num_cores=2)

    @functools.partial(
        pl.kernel,
        out_type=(jax.ShapeDtypeStruct((NP, G3), _F32),
                  jax.ShapeDtypeStruct((NP, GP), jnp.int32),
                  jax.ShapeDtypeStruct((NR, D), _F32)),
        mesh=mesh,
        scratch_types=[
            pltpu.VMEM((CH,), jnp.int32),
            pltpu.VMEM((CH, G3), _F32),
            pltpu.VMEM((CH, GP), jnp.int32),
            pltpu.VMEM((RPW,), jnp.int32),
            pltpu.VMEM((RPW, D), _F32),
            pltpu.SemaphoreType.DMA,
        ],
    )
    def k(ea_hbm, ebp_hbm, emb_hbm, ids_hbm, rids_hbm,
          ga_hbm, gbp_hbm, fr_hbm,
          idx_v, rows_v, rowsi_v, ridx_v, rrows_v, sem):
        cid = lax.axis_index("c")
        sid = lax.axis_index("s")
        wid = cid * 16 + sid

        def chunk(i, carry):
            base = wid * BPW + i * CH
            pltpu.sync_copy(ids_hbm.at[pl.ds(base, CH)], idx_v)
            pltpu.async_copy(ea_hbm.at[idx_v], rows_v, sem).wait()
            pltpu.sync_copy(rows_v, ga_hbm.at[pl.ds(base, CH)])
            pltpu.async_copy(ebp_hbm.at[idx_v], rowsi_v, sem).wait()
            pltpu.sync_copy(rowsi_v, gbp_hbm.at[pl.ds(base, CH)])
            return carry

        lax.fori_loop(0, NCH, chunk, 0)

        @pl.when(cid == 0)
        def _():
            rbase = sid * RPW
            pltpu.sync_copy(rids_hbm.at[pl.ds(rbase, RPW)], ridx_v)
            pltpu.async_copy(emb_hbm.at[ridx_v], rrows_v, sem).wait()
            pltpu.sync_copy(rrows_v, fr_hbm.at[pl.ds(rbase, RPW)])

    return k(ea, ebp, emb, ids_pad, rids_pad)


def _iteru_call(p, ga, whh, bhh2):
    N, G3 = ga.shape
    H = whh.shape[0]
    R = 2000

    def body(p_ref, ga_ref, whh_ref, bhh_ref, u_ref):
        s = p_ref[0] + p_ref[1]
        g = jnp.dot(s, whh_ref[...], preferred_element_type=_F32) + bhh_ref[...]
        ga_b = ga_ref[...]
        packed = _pack_bf16_cols(jnp.concatenate(
            [ga_b[:, :2 * H] + g[:, :2 * H], ga_b[:, 2 * H:], g[:, 2 * H:], s],
            axis=1))
        pad = jnp.zeros((packed.shape[0], 64), jnp.int32)
        u_ref[...] = jnp.concatenate([packed, pad], axis=1)

    UW = (G3 + 2 * H) // 2 + 64
    return pl.pallas_call(
        body,
        grid=(N // R,),
        in_specs=[
            pl.BlockSpec((2, R, H), lambda i: (0, i, 0)),
            pl.BlockSpec((R, G3), lambda i: (i, 0)),
            pl.BlockSpec((H, G3), lambda i: (0, 0)),
            pl.BlockSpec((1, G3), lambda i: (0, 0)),
        ],
        out_specs=pl.BlockSpec((R, UW), lambda i: (i, 0)),
        out_shape=jax.ShapeDtypeStruct((N, UW), jnp.int32),
    )(p, ga, whh, bhh2)


def _edge_call(u, gbp, src4, dst4, srct, dstt, zero_nh):
    N, UW = u.shape
    GP = gbp.shape[1]
    NW, NQ, _, K = src4.shape
    H = zero_nh.shape[1]
    NO = NQ // 2
    EL = H // 2
    mesh = plsc.VectorSubcoreMesh(core_axis_name="c", subcore_axis_name="s",
                                  num_cores=2)

    @functools.partial(
        pl.kernel,
        out_type=jax.ShapeDtypeStruct((2, N, H), _F32),
        mesh=mesh,
        scratch_types=[
            pltpu.VMEM_SHARED((N, H), _F32),
            pltpu.VMEM((4, K), jnp.int32),
            pltpu.VMEM((4, K), jnp.int32),
            pltpu.VMEM((4, K), jnp.int32),
            pltpu.VMEM((4, K), jnp.int32),
            pltpu.VMEM((4, K), jnp.int32),
            pltpu.VMEM((4, K), jnp.int32),
            pltpu.VMEM((1, K), jnp.int32),
            pltpu.VMEM((1, K), jnp.int32),
            [pltpu.VMEM((K, UW), jnp.int32) for _ in range(4)],
            [pltpu.VMEM((K, GP), jnp.int32) for _ in range(4)],
            [pltpu.VMEM((K, H), _F32) for _ in range(2)],
            [pltpu.SemaphoreType.DMA for _ in range(4)],
            [pltpu.SemaphoreType.DMA for _ in range(4)],
            [pltpu.SemaphoreType.DMA for _ in range(2)],
            [pltpu.SemaphoreType.DMA for _ in range(6)],
        ],
    )
    def k(u_hbm, gb_hbm, src_hbm, dst_hbm, srct_hbm, dstt_hbm, z_hbm, p_hbm,
          ssum, sA, sB, dGA, dGB, dSA, dSB, sT, dT,
          ubufs, bbufs, mbufs, semu, semb, semm, semi):
        (semSAs, semGAd, semSAd, semBs, semGBd, semSBd) = semi
        cid = lax.axis_index("c")
        sid = lax.axis_index("s")
        wid = cid * 16 + sid

        @pl.when(sid == 0)
        def _():
            pltpu.sync_copy(z_hbm, ssum)

        plsc.subcore_barrier()

        def gst(s_row, d_row, bi):
            pltpu.async_copy(u_hbm.at[s_row], ubufs[bi], semu[bi])
            pltpu.async_copy(gb_hbm.at[d_row], bbufs[bi], semb[bi])

        def gwait(bi):
            pltpu.make_async_copy(u_hbm.at[sT.at[0]], ubufs[bi],
                                  semu[bi]).wait()
            pltpu.make_async_copy(gb_hbm.at[dT.at[0]], bbufs[bi],
                                  semb[bi]).wait()

        def mdrain(mi):
            pltpu.make_async_copy(mbufs[mi], ssum.at[dT.at[0]],
                                  semm[mi]).wait()

        def cmp_(d_row, bi, mi):
            u_rows = ubufs[bi]
            b_rows = bbufs[bi]
            m_buf = mbufs[mi]

            def edge(e, ecarry):
                for g in range(H // 32):
                    ur = _unpk(u_rows[e, pl.ds(16 * g, 16)])
                    uz = _unpk(u_rows[e, pl.ds(EL + 16 * g, 16)])
                    un = _unpk(u_rows[e, pl.ds(2 * EL + 16 * g, 16)])
                    uhn = _unpk(u_rows[e, pl.ds(3 * EL + 16 * g, 16)])
                    uh = _unpk(u_rows[e, pl.ds(4 * EL + 16 * g, 16)])
                    br = _unpk(b_rows[e, pl.ds(16 * g, 16)])
                    bz = _unpk(b_rows[e, pl.ds(EL + 16 * g, 16)])
                    bn = _unpk(b_rows[e, pl.ds(2 * EL + 16 * g, 16)])
                    for hf in range(2):
                        r = _sigmoid16(ur[hf] + br[hf])
                        z = _sigmoid16(uz[hf] + bz[hf])
                        nn = _tanh16(un[hf] + bn[hf] + r * uhn[hf])
                        m_buf[e, pl.ds(32 * g + 16 * hf, 16)] = (
                            (1.0 - z) * nn + z * uh[hf])
                return ecarry

            lax.fori_loop(0, K, edge, 0)
            pltpu.async_copy(m_buf, ssum.at[d_row], semm[mi], add=True)

        def islab_load(q, slab, sem, which):
            hbm = src_hbm if which == 0 else dst_hbm
            pltpu.async_copy(hbm.at[wid, q], slab, sem)

        def islab_wait(slab, sem, which):
            hbm = src_hbm if which == 0 else dst_hbm
            pltpu.make_async_copy(hbm.at[wid, 0], slab, sem).wait()

        pltpu.sync_copy(src_hbm.at[wid, 0], sA)
        pltpu.sync_copy(dst_hbm.at[wid, 0], dGA)
        pltpu.sync_copy(dst_hbm.at[wid, 0], dSA)
        pltpu.sync_copy(src_hbm.at[wid, 1], sB)
        pltpu.sync_copy(dst_hbm.at[wid, 1], dGB)
        pltpu.sync_copy(dst_hbm.at[wid, 1], dSB)
        pltpu.sync_copy(srct_hbm.at[wid], sT)
        pltpu.sync_copy(dstt_hbm.at[wid], dT)
        gst(sA.at[0], dGA.at[0], 0)
        gst(sA.at[1], dGA.at[1], 1)
        gst(sA.at[2], dGA.at[2], 2)

        def oct_(o, carry):
            nxt = o + 1 < NO
            lat = o > 0
            @pl.when(lat)
            def _():
                islab_wait(dSA, semSAd, 1)
            gst(sA.at[3], dGA.at[3], 3)
            gwait(0)
            @pl.when(lat)
            def _():
                mdrain(0)
            cmp_(dSA.at[0], 0, 0)
            @pl.when(nxt)
            def _():
                islab_load(2 * o + 2, sA, semSAs, 0)
                islab_load(2 * o + 2, dGA, semGAd, 1)
            @pl.when(lat)
            def _():
                islab_wait(sB, semBs, 0)
                islab_wait(dGB, semGBd, 1)
            gst(sB.at[0], dGB.at[0], 0)
            gwait(1)
            @pl.when(lat)
            def _():
                mdrain(1)
            cmp_(dSA.at[1], 1, 1)
            gst(sB.at[1], dGB.at[1], 1)
            gwait(2)
            mdrain(0)
            cmp_(dSA.at[2], 2, 0)
            @pl.when(lat)
            def _():
                islab_load(2 * o + 1, dSB, semSBd, 1)
            gst(sB.at[2], dGB.at[2], 2)
            gwait(3)
            mdrain(1)
            cmp_(dSA.at[3], 3, 1)
            @pl.when(lat)
            def _():
                islab_wait(dSB, semSBd, 1)
            gst(sB.at[3], dGB.at[3], 3)
            gwait(0)
            mdrain(0)
            cmp_(dSB.at[0], 0, 0)
            @pl.when(nxt)
            def _():
                islab_load(2 * o + 3, sB, semBs, 0)
                islab_load(2 * o + 3, dGB, semGBd, 1)
            @pl.when(nxt)
            def _():
                islab_wait(sA, semSAs, 0)
                islab_wait(dGA, semGAd, 1)
                gst(sA.at[0], dGA.at[0], 0)
            gwait(1)
            mdrain(1)
            cmp_(dSB.at[1], 1, 1)
            @pl.when(nxt)
            def _():
                islab_load(2 * o + 2, dSA, semSAd, 1)
            @pl.when(nxt)
            def _():
                gst(sA.at[1], dGA.at[1], 1)
            gwait(2)
            mdrain(0)
            cmp_(dSB.at[2], 2, 0)
            @pl.when(nxt)
            def _():
                gst(sA.at[2], dGA.at[2], 2)
            gwait(3)
            mdrain(1)
            cmp_(dSB.at[3], 3, 1)
            return carry

        lax.fori_loop(0, NO, oct_, 0)

        gst(sT.at[0], dT.at[0], 0)
        gwait(0)
        mdrain(0)
        cmp_(dT.at[0], 0, 0)
        mdrain(1)
        mdrain(0)

        plsc.subcore_barrier()

        @pl.when(sid == 0)
        def _():
            pltpu.sync_copy(ssum, p_hbm.at[cid])

    return k(u, gbp, src4, dst4, srct, dstt, zero_nh)


def _final_call(fr, pr):
    R, D = fr.shape
    H = pr.shape[2]

    def body(fr_ref, pr_ref, o_ref):
        o_ref[...] = jnp.concatenate([fr_ref[...], pr_ref[0] + pr_ref[1]],
                                     axis=1)

    return pl.pallas_call(
        body,
        out_shape=jax.ShapeDtypeStruct((R, D + H), _F32),
    )(fr, pr)


def kernel(ids, edge_index, depth, embeddings, W_ih, W_hh, b_ih, b_hh):
    N = ids.shape[0]
    V, D = embeddings.shape
    H = W_hh.shape[0]
    src = edge_index[0].astype(jnp.int32)
    dst = edge_index[1].astype(jnp.int32)

    ea, ebp = _prep_call(embeddings, W_ih, b_ih.reshape(1, -1))

    ids32 = ids.astype(jnp.int32)
    NP = 10240
    ids_pad = jnp.concatenate([ids32, jnp.zeros((NP - N,), jnp.int32)])
    rids = ids32.reshape(100, 100)[:, 0]
    rids_pad = jnp.concatenate([rids, jnp.zeros((28,), jnp.int32)])
    ga_p, gbp_p, fr = _gather_call(ea, ebp, embeddings, ids_pad, rids_pad)
    ga = ga_p[:N]
    gbp = gbp_p[:N]

    zero_nh = jnp.zeros((N, H), _F32)
    p0 = jnp.zeros((2, N, H), _F32)
    bhh2 = b_hh.reshape(1, -1)

    NW, K = 32, 16
    EPW = src.shape[0] // NW
    NB = EPW // K
    NQB = (NB - 1) // 4
    src_w = src.reshape(NW, NB, K)
    dst_w = dst.reshape(NW, NB, K)
    src4 = src_w[:, :4 * NQB].reshape(NW, NQB, 4, K)
    dst4 = dst_w[:, :4 * NQB].reshape(NW, NQB, 4, K)
    srct = src_w[:, 4 * NQB:]
    dstt = dst_w[:, 4 * NQB:]

    def body(_, p):
        uu = _iteru_call(p, ga, W_hh, bhh2)
        return _edge_call(uu, gbp, src4, dst4, srct, dstt, zero_nh)

    p = lax.fori_loop(0, depth, body, p0)

    pr = p[:, ::100, :]
    return _final_call(fr[:100], pr)
